# baseline (speedup 1.0000x reference)
import jax
import jax.numpy as jnp
from jax import lax
from jax.experimental import pallas as pl
from jax.experimental.pallas import tpu as pltpu
from jax.experimental.pallas import tpu_sc as plsc

B = 128
S = 64
V = 50000
WD = 300
CV = 100
CD = 15
LW = 16
NF = 100
DE = 300
LH = 300
NC = 4

NTOK = 3 * B * S
XW = 640
NW = 32
ROWS_PER_W = NTOK // NW
CHUNK = 128
NCHUNK = ROWS_PER_W // CHUNK


def _sc_gather2_body(tab_hbm, idx_hbm, out_hbm, idx_v, rows_v, sem):
    wid = lax.axis_index("s") * 2 + lax.axis_index("c")
    base = wid * ROWS_PER_W
    pltpu.sync_copy(idx_hbm.at[pl.ds(base, ROWS_PER_W)], idx_v)

    def chunk_body(i, _):
        off = i * CHUNK
        ix = idx_v.at[pl.ds(off, CHUNK)]
        c0 = pltpu.async_copy(tab_hbm.at[ix, pl.ds(0, 128)],
                              rows_v.at[:, pl.ds(0, 128)], sem)
        c1 = pltpu.async_copy(tab_hbm.at[ix, pl.ds(128, 128)],
                              rows_v.at[:, pl.ds(128, 128)], sem)
        c0.wait()
        c1.wait()
        pltpu.sync_copy(rows_v, out_hbm.at[pl.ds(base + off, CHUNK)])
        return ()

    lax.fori_loop(0, NCHUNK, chunk_body, ())


def _sc_gather1_body(tab_hbm, idx_hbm, out_hbm, idx_v, rows_v, sem):
    wid = lax.axis_index("s") * 2 + lax.axis_index("c")
    base = wid * ROWS_PER_W
    pltpu.sync_copy(idx_hbm.at[pl.ds(base, ROWS_PER_W)], idx_v)

    def chunk_body(i, _):
        off = i * CHUNK
        ix = idx_v.at[pl.ds(off, CHUNK)]
        pltpu.async_copy(tab_hbm.at[ix], rows_v, sem).wait()
        pltpu.sync_copy(rows_v, out_hbm.at[pl.ds(base + off, CHUNK)])
        return ()

    lax.fori_loop(0, NCHUNK, chunk_body, ())


def _sc_gather2(tab, idx):
    mesh = plsc.VectorSubcoreMesh(core_axis_name="c", subcore_axis_name="s")
    f = pl.kernel(
        _sc_gather2_body,
        out_type=jax.ShapeDtypeStruct((NTOK, 256), jnp.float32),
        mesh=mesh,
        scratch_types=[
            pltpu.VMEM((ROWS_PER_W,), jnp.int32),
            pltpu.VMEM((CHUNK, 256), jnp.float32),
            pltpu.SemaphoreType.DMA,
        ],
    )
    return f(tab, idx)


def _sc_gather1(tab, idx):
    mesh = plsc.VectorSubcoreMesh(core_axis_name="c", subcore_axis_name="s")
    f = pl.kernel(
        _sc_gather1_body,
        out_type=jax.ShapeDtypeStruct((NTOK, 128), jnp.float32),
        mesh=mesh,
        scratch_types=[
            pltpu.VMEM((ROWS_PER_W,), jnp.int32),
            pltpu.VMEM((CHUNK, 128), jnp.float32),
            pltpu.SemaphoreType.DMA,
        ],
    )
    return f(tab, idx)


TROWS = 2000


def _tail_body(g_ref, w_ref, o_ref):
    z = jnp.zeros((TROWS, 128 - 2 * (WD - 256)), jnp.float32)
    o_ref[:, :] = jnp.concatenate(
        [g_ref[:, 0:WD - 256], w_ref[:, 0:WD - 256], z], axis=1)


def _tail_pack(glove_w, w2v_w):
    return pl.pallas_call(
        _tail_body,
        grid=(V // TROWS,),
        in_specs=[pl.BlockSpec((TROWS, 128), lambda i: (i, 2)),
                  pl.BlockSpec((TROWS, 128), lambda i: (i, 2))],
        out_specs=pl.BlockSpec((TROWS, 128), lambda i: (i, 0)),
        out_shape=jax.ShapeDtypeStruct((V, 128), jnp.float32),
    )(glove_w, w2v_w)


BB = 16
TOK = BB * S
NBB = B // BB


def _enc_body(xg_ref, xw_ref, xt_ref, ch_ref, mask_ref, cnt_ref,
              char_w_ref, conv_w_ref, conv_b_ref, wp_ref, bp_ref,
              u_ref):
    ch = ch_ref[0].reshape(TOK * LW, 1)
    oh = (ch == lax.broadcasted_iota(jnp.int32, (1, CV), 1)).astype(jnp.float32)
    e = jnp.dot(oh, char_w_ref[:, :], preferred_element_type=jnp.float32)
    q0 = jnp.dot(e, conv_w_ref[0], preferred_element_type=jnp.float32)
    q1 = jnp.dot(e, conv_w_ref[1], preferred_element_type=jnp.float32)
    q2 = jnp.dot(e, conv_w_ref[2], preferred_element_type=jnp.float32)
    q0 = q0.reshape(TOK, LW, NF)
    q1 = q1.reshape(TOK, LW, NF)
    q2 = q2.reshape(TOK, LW, NF)
    y = q0[:, 0:LW - 2, :] + q1[:, 1:LW - 1, :] + q2[:, 2:LW, :]
    y = jnp.maximum(y + conv_b_ref[:, :], 0.0)
    c = jnp.max(y, axis=1)

    bf = jnp.bfloat16
    xg = xg_ref[:, :].astype(bf)
    xw = xw_ref[:, :].astype(bf)
    xt = xt_ref[:, :].astype(bf)
    wp = wp_ref[:, :].astype(bf)
    h = (jnp.dot(xg, wp[0:256], preferred_element_type=jnp.float32)
         + jnp.dot(xw, wp[WD:WD + 256], preferred_element_type=jnp.float32)
         + jnp.dot(xt[:, 0:WD - 256], wp[256:WD], preferred_element_type=jnp.float32)
         + jnp.dot(xt[:, 44:44 + (WD - 256)], wp[WD + 256:2 * WD], preferred_element_type=jnp.float32)
         + jnp.dot(c.astype(bf), wp[2 * WD:2 * WD + NF], preferred_element_type=jnp.float32)
         + bp_ref[:, :])
    h = jnp.maximum(h, 0.0)

    m = mask_ref[0]
    hm = (h * m).reshape(BB, S, 2 * DE)
    hx = jnp.where(m > 0.0, h, -1e9).reshape(BB, S, 2 * DE)
    mean = jnp.sum(hm, axis=1) / cnt_ref[0]
    mx = jnp.max(hx, axis=1)
    u_ref[0] = jnp.concatenate([mean, mx], axis=-1)


def _tc_encode(xg_all, xw_all, xt_all, chars_all, mask_all, cnt_all,
               char_w, conv_w, conv_b, Wp, bp):
    grid = (3 * NBB,)
    return pl.pallas_call(
        _enc_body,
        grid=grid,
        in_specs=[
            pl.BlockSpec((TOK, 256), lambda i: (i, 0)),
            pl.BlockSpec((TOK, 256), lambda i: (i, 0)),
            pl.BlockSpec((TOK, 128), lambda i: (i, 0)),
            pl.BlockSpec((1, 1, TOK * LW), lambda i: (i, 0, 0)),
            pl.BlockSpec((1, TOK, 1), lambda i: (i, 0, 0)),
            pl.BlockSpec((1, BB, 1), lambda i: (i, 0, 0)),
            pl.BlockSpec((CV, CD), lambda i: (0, 0)),
            pl.BlockSpec((3, CD, NF), lambda i: (0, 0, 0)),
            pl.BlockSpec((1, NF), lambda i: (0, 0)),
            pl.BlockSpec((2 * WD + NF, 2 * DE), lambda i: (0, 0)),
            pl.BlockSpec((1, 2 * DE), lambda i: (0, 0)),
        ],
        out_specs=pl.BlockSpec((1, BB, 4 * DE), lambda i: (i, 0, 0)),
        out_shape=jax.ShapeDtypeStruct((3 * NBB, BB, 4 * DE), jnp.float32),
    )(xg_all, xw_all, xt_all, chars_all, mask_all, cnt_all,
      char_w, conv_w, conv_b.reshape(1, NF), Wp,
      bp.reshape(1, 2 * DE))


def _head_body(u_ref, wihf_ref, whhf_ref, bf_ref, wihb_ref, whhb_ref, bb_ref,
               w1_ref, b1_ref, w2_ref, b2_ref, wo_ref, bo_ref, out_ref):
    u1 = u_ref[0]
    u2 = u_ref[1]
    u3 = u_ref[2]

    def lstm(xs, wih_ref, whh_ref, b_ref):
        h = jnp.zeros((B, LH), jnp.float32)
        c = jnp.zeros((B, LH), jnp.float32)
        for x in xs:
            z = (jnp.dot(x, wih_ref[:, :], preferred_element_type=jnp.float32)
                 + jnp.dot(h, whh_ref[:, :], preferred_element_type=jnp.float32)
                 + b_ref[:, :])
            i = jax.nn.sigmoid(z[:, 0 * LH:1 * LH])
            f = jax.nn.sigmoid(z[:, 1 * LH:2 * LH])
            g = jnp.tanh(z[:, 2 * LH:3 * LH])
            o = jax.nn.sigmoid(z[:, 3 * LH:4 * LH])
            c = f * c + i * g
            h = o * jnp.tanh(c)
        return h

    hf = lstm([u1, u2, u3], wihf_ref, whhf_ref, bf_ref)
    hb = lstm([u3, u2, u1], wihb_ref, whhb_ref, bb_ref)

    u = jnp.concatenate([u1, u2, u3, u1 - u2 + u3, hf, hb], axis=-1)
    o1 = jnp.maximum(jnp.dot(u, w1_ref[:, :], preferred_element_type=jnp.float32)
                     + b1_ref[:, :], 0.0)
    o2 = (jnp.dot(u, w2_ref[0:16 * DE + 2 * LH, :], preferred_element_type=jnp.float32)
          + jnp.dot(o1, w2_ref[16 * DE + 2 * LH:, :], preferred_element_type=jnp.float32)
          + b2_ref[:, :])
    o2 = jnp.maximum(o2, 0.0)
    out_ref[:, :] = (jnp.dot(o2, wo_ref[:, :], preferred_element_type=jnp.float32)
                     + bo_ref[:, :])


def _tc_head(u_stack, Wih_f, Whh_f, b_f, Wih_b, Whh_b, b_b,
             W1, b1, W2, b2, Wo, bo):
    return pl.pallas_call(
        _head_body,
        out_shape=jax.ShapeDtypeStruct((B, NC), jnp.float32),
    )(u_stack, Wih_f, Whh_f, b_f.reshape(1, -1), Wih_b, Whh_b,
      b_b.reshape(1, -1), W1, b1.reshape(1, -1), W2, b2.reshape(1, -1),
      Wo, bo.reshape(1, -1))


def kernel(seq_turn1, seq_turn2, seq_turn3, lens_turn1, lens_turn2, lens_turn3,
           char_turn1, char_turn2, char_turn3,
           glove_w, w2v_w, char_w, conv_w, conv_b, Wp, bp,
           Wih_f, Whh_f, b_f, Wih_b, Whh_b, b_b,
           W1, b1, W2, b2, Wo, bo):
    idx = jnp.concatenate([seq_turn1.reshape(-1), seq_turn2.reshape(-1),
                           seq_turn3.reshape(-1)]).astype(jnp.int32)

    xg_all = _sc_gather2(glove_w, idx)
    xw_all = _sc_gather2(w2v_w, idx)
    tail_cat = _tail_pack(glove_w, w2v_w)
    xt_all = _sc_gather1(tail_cat, idx)

    chars_all = jnp.stack([char_turn1, char_turn2, char_turn3]) \
        .astype(jnp.int32).reshape(3 * NBB, 1, TOK * LW)

    lens = jnp.stack([lens_turn1, lens_turn2, lens_turn3])
    pos = lax.broadcasted_iota(jnp.int32, (1, B, S), 2)
    mask = (pos < lens[:, :, None]).astype(jnp.float32)
    mask_all = mask.reshape(3 * NBB, TOK, 1)
    cnt_all = jnp.maximum(jnp.sum(mask, axis=2), 1.0) \
        .reshape(3 * NBB, BB, 1)

    u_blocks = _tc_encode(xg_all, xw_all, xt_all, chars_all, mask_all,
                          cnt_all, char_w, conv_w, conv_b, Wp, bp)
    u_stack = u_blocks.reshape(3, B, 4 * DE)

    return _tc_head(u_stack, Wih_f, Whh_f, b_f, Wih_b, Whh_b, b_b,
                    W1, b1, W2, b2, Wo, bo)


# speedup vs baseline: 1.0098x; 1.0098x over previous
import jax
import jax.numpy as jnp
from jax import lax
from jax.experimental import pallas as pl
from jax.experimental.pallas import tpu as pltpu
from jax.experimental.pallas import tpu_sc as plsc

B = 128
S = 64
V = 50000
WD = 300
CV = 100
CD = 15
LW = 16
NF = 100
DE = 300
LH = 300
NC = 4

NTOK = 3 * B * S
XW = 640
NW = 32
ROWS_PER_W = NTOK // NW
CHUNK = 128
NCHUNK = ROWS_PER_W // CHUNK


def _sc_gather_body(glove_hbm, w2v_hbm, tail_hbm, idx_hbm, out_hbm,
                    idx_v, rows_v, sem):
    wid = lax.axis_index("s") * 2 + lax.axis_index("c")
    base = wid * ROWS_PER_W
    pltpu.sync_copy(idx_hbm.at[pl.ds(base, ROWS_PER_W)], idx_v)

    def chunk_body(i, _):
        off = i * CHUNK
        ix = idx_v.at[pl.ds(off, CHUNK)]
        c0 = pltpu.async_copy(glove_hbm.at[ix, pl.ds(0, 128)],
                              rows_v.at[:, pl.ds(0, 128)], sem)
        c1 = pltpu.async_copy(glove_hbm.at[ix, pl.ds(128, 128)],
                              rows_v.at[:, pl.ds(128, 128)], sem)
        c2 = pltpu.async_copy(w2v_hbm.at[ix, pl.ds(0, 128)],
                              rows_v.at[:, pl.ds(256, 128)], sem)
        c3 = pltpu.async_copy(w2v_hbm.at[ix, pl.ds(128, 128)],
                              rows_v.at[:, pl.ds(384, 128)], sem)
        c4 = pltpu.async_copy(tail_hbm.at[ix], rows_v.at[:, pl.ds(512, 128)],
                              sem)
        c0.wait()
        c1.wait()
        c2.wait()
        c3.wait()
        c4.wait()
        pltpu.sync_copy(rows_v, out_hbm.at[pl.ds(base + off, CHUNK)])
        return ()

    lax.fori_loop(0, NCHUNK, chunk_body, ())


def _sc_gather(glove_w, w2v_w, tail_cat, idx):
    mesh = plsc.VectorSubcoreMesh(core_axis_name="c", subcore_axis_name="s")
    f = pl.kernel(
        _sc_gather_body,
        out_type=jax.ShapeDtypeStruct((NTOK, XW), jnp.float32),
        mesh=mesh,
        scratch_types=[
            pltpu.VMEM((ROWS_PER_W,), jnp.int32),
            pltpu.VMEM((CHUNK, XW), jnp.float32),
            pltpu.SemaphoreType.DMA,
        ],
    )
    return f(glove_w, w2v_w, tail_cat, idx)


TROWS = 2000


def _tail_body(g_ref, w_ref, o_ref):
    z = jnp.zeros((TROWS, 128 - 2 * (WD - 256)), jnp.float32)
    o_ref[:, :] = jnp.concatenate(
        [g_ref[:, 0:WD - 256], w_ref[:, 0:WD - 256], z], axis=1)


def _tail_pack(glove_w, w2v_w):
    return pl.pallas_call(
        _tail_body,
        grid=(V // TROWS,),
        in_specs=[pl.BlockSpec((TROWS, 128), lambda i: (i, 2)),
                  pl.BlockSpec((TROWS, 128), lambda i: (i, 2))],
        out_specs=pl.BlockSpec((TROWS, 128), lambda i: (i, 0)),
        out_shape=jax.ShapeDtypeStruct((V, 128), jnp.float32),
    )(glove_w, w2v_w)


BB = 16
TOK = BB * S
NBB = B // BB


def _enc_body(x_ref, ch_ref, mask_ref, cnt_ref,
              char_w_ref, conv_w_ref, conv_b_ref, wp_ref, bp_ref,
              u_ref):
    ch = ch_ref[0].reshape(TOK * LW, 1)
    oh = (ch == lax.broadcasted_iota(jnp.int32, (1, CV), 1)).astype(jnp.float32)
    e = jnp.dot(oh, char_w_ref[:, :], preferred_element_type=jnp.float32)
    q0 = jnp.dot(e, conv_w_ref[0], preferred_element_type=jnp.float32)
    q1 = jnp.dot(e, conv_w_ref[1], preferred_element_type=jnp.float32)
    q2 = jnp.dot(e, conv_w_ref[2], preferred_element_type=jnp.float32)
    q0 = q0.reshape(TOK, LW, NF)
    q1 = q1.reshape(TOK, LW, NF)
    q2 = q2.reshape(TOK, LW, NF)
    y = q0[:, 0:LW - 2, :] + q1[:, 1:LW - 1, :] + q2[:, 2:LW, :]
    y = jnp.maximum(y + conv_b_ref[:, :], 0.0)
    c = jnp.max(y, axis=1)

    bf = jnp.bfloat16
    x = x_ref[:, :].astype(bf)
    wp = wp_ref[:, :].astype(bf)
    h = (jnp.dot(x[:, 0:256], wp[0:256], preferred_element_type=jnp.float32)
         + jnp.dot(x[:, 256:512], wp[WD:WD + 256], preferred_element_type=jnp.float32)
         + jnp.dot(x[:, 512:512 + (WD - 256)], wp[256:WD], preferred_element_type=jnp.float32)
         + jnp.dot(x[:, 556:556 + (WD - 256)], wp[WD + 256:2 * WD], preferred_element_type=jnp.float32)
         + jnp.dot(c.astype(bf), wp[2 * WD:2 * WD + NF], preferred_element_type=jnp.float32)
         + bp_ref[:, :])
    h = jnp.maximum(h, 0.0)

    m = mask_ref[0]
    hm = (h * m).reshape(BB, S, 2 * DE)
    hx = jnp.where(m > 0.0, h, -1e9).reshape(BB, S, 2 * DE)
    mean = jnp.sum(hm, axis=1) / cnt_ref[0]
    mx = jnp.max(hx, axis=1)
    u_ref[0] = jnp.concatenate([mean, mx], axis=-1)


def _tc_encode(x_all, chars_all, mask_all, cnt_all,
               char_w, conv_w, conv_b, Wp, bp):
    grid = (3 * NBB,)
    return pl.pallas_call(
        _enc_body,
        grid=grid,
        in_specs=[
            pl.BlockSpec((TOK, XW), lambda i: (i, 0)),
            pl.BlockSpec((1, 1, TOK * LW), lambda i: (i, 0, 0)),
            pl.BlockSpec((1, TOK, 1), lambda i: (i, 0, 0)),
            pl.BlockSpec((1, BB, 1), lambda i: (i, 0, 0)),
            pl.BlockSpec((CV, CD), lambda i: (0, 0)),
            pl.BlockSpec((3, CD, NF), lambda i: (0, 0, 0)),
            pl.BlockSpec((1, NF), lambda i: (0, 0)),
            pl.BlockSpec((2 * WD + NF, 2 * DE), lambda i: (0, 0)),
            pl.BlockSpec((1, 2 * DE), lambda i: (0, 0)),
        ],
        out_specs=pl.BlockSpec((1, BB, 4 * DE), lambda i: (i, 0, 0)),
        out_shape=jax.ShapeDtypeStruct((3 * NBB, BB, 4 * DE), jnp.float32),
    )(x_all, chars_all, mask_all, cnt_all,
      char_w, conv_w, conv_b.reshape(1, NF), Wp,
      bp.reshape(1, 2 * DE))


def _head_body(u_ref, wihf_ref, whhf_ref, bf_ref, wihb_ref, whhb_ref, bb_ref,
               w1_ref, b1_ref, w2_ref, b2_ref, wo_ref, bo_ref, out_ref):
    u1 = u_ref[0]
    u2 = u_ref[1]
    u3 = u_ref[2]

    def lstm(xs, wih_ref, whh_ref, b_ref):
        h = jnp.zeros((B, LH), jnp.float32)
        c = jnp.zeros((B, LH), jnp.float32)
        for x in xs:
            z = (jnp.dot(x, wih_ref[:, :], preferred_element_type=jnp.float32)
                 + jnp.dot(h, whh_ref[:, :], preferred_element_type=jnp.float32)
                 + b_ref[:, :])
            i = jax.nn.sigmoid(z[:, 0 * LH:1 * LH])
            f = jax.nn.sigmoid(z[:, 1 * LH:2 * LH])
            g = jnp.tanh(z[:, 2 * LH:3 * LH])
            o = jax.nn.sigmoid(z[:, 3 * LH:4 * LH])
            c = f * c + i * g
            h = o * jnp.tanh(c)
        return h

    hf = lstm([u1, u2, u3], wihf_ref, whhf_ref, bf_ref)
    hb = lstm([u3, u2, u1], wihb_ref, whhb_ref, bb_ref)

    u = jnp.concatenate([u1, u2, u3, u1 - u2 + u3, hf, hb], axis=-1)
    o1 = jnp.maximum(jnp.dot(u, w1_ref[:, :], preferred_element_type=jnp.float32)
                     + b1_ref[:, :], 0.0)
    o2 = (jnp.dot(u, w2_ref[0:16 * DE + 2 * LH, :], preferred_element_type=jnp.float32)
          + jnp.dot(o1, w2_ref[16 * DE + 2 * LH:, :], preferred_element_type=jnp.float32)
          + b2_ref[:, :])
    o2 = jnp.maximum(o2, 0.0)
    out_ref[:, :] = (jnp.dot(o2, wo_ref[:, :], preferred_element_type=jnp.float32)
                     + bo_ref[:, :])


def _tc_head(u_stack, Wih_f, Whh_f, b_f, Wih_b, Whh_b, b_b,
             W1, b1, W2, b2, Wo, bo):
    return pl.pallas_call(
        _head_body,
        out_shape=jax.ShapeDtypeStruct((B, NC), jnp.float32),
    )(u_stack, Wih_f, Whh_f, b_f.reshape(1, -1), Wih_b, Whh_b,
      b_b.reshape(1, -1), W1, b1.reshape(1, -1), W2, b2.reshape(1, -1),
      Wo, bo.reshape(1, -1))


def kernel(seq_turn1, seq_turn2, seq_turn3, lens_turn1, lens_turn2, lens_turn3,
           char_turn1, char_turn2, char_turn3,
           glove_w, w2v_w, char_w, conv_w, conv_b, Wp, bp,
           Wih_f, Whh_f, b_f, Wih_b, Whh_b, b_b,
           W1, b1, W2, b2, Wo, bo):
    idx = jnp.concatenate([seq_turn1.reshape(-1), seq_turn2.reshape(-1),
                           seq_turn3.reshape(-1)]).astype(jnp.int32)

    tail_cat = _tail_pack(glove_w, w2v_w)

    x_all = _sc_gather(glove_w, w2v_w, tail_cat, idx)

    chars_all = jnp.stack([char_turn1, char_turn2, char_turn3]) \
        .astype(jnp.int32).reshape(3 * NBB, 1, TOK * LW)

    lens = jnp.stack([lens_turn1, lens_turn2, lens_turn3])
    pos = lax.broadcasted_iota(jnp.int32, (1, B, S), 2)
    mask = (pos < lens[:, :, None]).astype(jnp.float32)
    mask_all = mask.reshape(3 * NBB, TOK, 1)
    cnt_all = jnp.maximum(jnp.sum(mask, axis=2), 1.0) \
        .reshape(3 * NBB, BB, 1)

    u_blocks = _tc_encode(x_all, chars_all, mask_all, cnt_all,
                          char_w, conv_w, conv_b, Wp, bp)
    u_stack = u_blocks.reshape(3, B, 4 * DE)

    return _tc_head(u_stack, Wih_f, Whh_f, b_f, Wih_b, Whh_b, b_b,
                    W1, b1, W2, b2, Wo, bo)
